# baseline (speedup 1.0000x reference)
import functools

import jax
import jax.numpy as jnp
from jax import lax
from jax.experimental import pallas as pl
from jax.experimental.pallas import tpu as pltpu
from jax.experimental.pallas import tpu_sc as plsc

VOCAB = 1000000
DIM = 64
BATCH = 16384

_info = plsc.get_sparse_core_info()
_NC, _NS, _L = _info.num_cores, _info.num_subcores, _info.num_lanes
_NW = _NC * _NS
_BPW = BATCH // _NW
_GROUPS = _BPW // _L
_CHUNKS = DIM // _L
_TSTRIDE = _L + 1


def _sc_body(tw_hbm, cw_hbm, ttab_hbm, ctab_hbm, out_hbm,
             idx_t, idx_c, rows_t, rows_c, acc, out_v, sem_t, sem_c):
    wid = lax.axis_index("s") * _NC + lax.axis_index("c")
    base = wid * _BPW

    pltpu.sync_copy(tw_hbm.at[pl.ds(base, _BPW)], idx_t)
    pltpu.sync_copy(cw_hbm.at[pl.ds(base, _BPW)], idx_c)
    cp_t = pltpu.async_copy(ttab_hbm.at[idx_t], rows_t, sem_t)
    cp_c = pltpu.async_copy(ctab_hbm.at[idx_c], rows_c, sem_c)
    cp_t.wait()
    cp_c.wait()

    lanes = lax.iota(jnp.int32, _L)

    def group(g, carry):
        for r in range(_L):
            row = g * _L + r
            p = rows_t[row, pl.ds(0, _L)] * rows_c[row, pl.ds(0, _L)]
            for j in range(1, _CHUNKS):
                p = p + rows_t[row, pl.ds(j * _L, _L)] * rows_c[row, pl.ds(j * _L, _L)]
            plsc.store_scatter(acc, [r * _TSTRIDE + lanes], p)
        s = plsc.load_gather(acc, [lanes * _TSTRIDE])
        for l in range(1, _L):
            s = s + plsc.load_gather(acc, [lanes * _TSTRIDE + l])
        sig = 1.0 / (1.0 + jnp.exp(-s))
        out_v[pl.ds(g * _L, _L)] = sig
        return carry

    lax.fori_loop(0, _GROUPS, group, 0)
    pltpu.sync_copy(out_v, out_hbm.at[pl.ds(base, _BPW)])


@functools.partial(jax.jit, static_argnames=())
def _run(tw, cw, ttab, ctab):
    mesh = plsc.VectorSubcoreMesh(core_axis_name="c", subcore_axis_name="s")
    kern = functools.partial(
        pl.kernel,
        mesh=mesh,
        compiler_params=pltpu.CompilerParams(
            needs_layout_passes=False, use_tc_tiling_on_sc=False
        ),
        out_type=jax.ShapeDtypeStruct((BATCH,), jnp.float32),
        scratch_types=[
            pltpu.VMEM((_BPW,), jnp.int32),
            pltpu.VMEM((_BPW,), jnp.int32),
            pltpu.VMEM((_BPW, DIM), jnp.float32),
            pltpu.VMEM((_BPW, DIM), jnp.float32),
            pltpu.VMEM((_L * _TSTRIDE,), jnp.float32),
            pltpu.VMEM((_BPW,), jnp.float32),
            pltpu.SemaphoreType.DMA,
            pltpu.SemaphoreType.DMA,
        ],
    )(_sc_body)
    return kern(tw, cw, ttab, ctab)


def kernel(target_word, context_word, target_table, context_table):
    tw = target_word.astype(jnp.int32)
    cw = context_word.astype(jnp.int32)
    return _run(tw, cw, target_table, context_table)


# speedup vs baseline: 1.4645x; 1.4645x over previous
import functools

import jax
import jax.numpy as jnp
from jax import lax
from jax.experimental import pallas as pl
from jax.experimental.pallas import tpu as pltpu
from jax.experimental.pallas import tpu_sc as plsc

VOCAB = 1000000
DIM = 64
BATCH = 16384

_info = plsc.get_sparse_core_info()
_NC, _NS, _L = _info.num_cores, _info.num_subcores, _info.num_lanes
_NW = _NC * _NS
_BPW = BATCH // _NW
_GROUPS = _BPW // _L
_CHUNKS = DIM // _L
_TSTRIDE = _L + 1
_TILES = VOCAB // 8


def _sc_body(tw_hbm, cw_hbm, ttab_hbm, ctab_hbm, out_hbm,
             idx_t, idx_c, buf_t, buf_c, acc, out_v, sem_t, sem_c):
    wid = lax.axis_index("s") * _NC + lax.axis_index("c")
    base = wid * _BPW

    pltpu.sync_copy(tw_hbm.at[pl.ds(base, _BPW)], idx_t)
    pltpu.sync_copy(cw_hbm.at[pl.ds(base, _BPW)], idx_c)

    tt3 = ttab_hbm.reshape(_TILES, 8, DIM)
    ct3 = ctab_hbm.reshape(_TILES, 8, DIM)
    lanes = lax.iota(jnp.int32, _L)

    def group(g, carry):
        ivt = idx_t[pl.ds(g * _L, _L)]
        ivc = idx_c[pl.ds(g * _L, _L)]
        copies = []
        for r in range(_L):
            et = ivt[r]
            ec = ivc[r]
            copies.append(pltpu.async_copy(
                tt3.at[et >> 3], buf_t.at[r], sem_t))
            copies.append(pltpu.async_copy(
                ct3.at[ec >> 3], buf_c.at[r], sem_c))
        for cp in copies:
            cp.wait()
        for r in range(_L):
            st = ivt[r] & 7
            sc_ = ivc[r] & 7
            p = buf_t[r, st, pl.ds(0, _L)] * buf_c[r, sc_, pl.ds(0, _L)]
            for j in range(1, _CHUNKS):
                p = p + (buf_t[r, st, pl.ds(j * _L, _L)]
                         * buf_c[r, sc_, pl.ds(j * _L, _L)])
            plsc.store_scatter(acc, [r * _TSTRIDE + lanes], p)
        s = plsc.load_gather(acc, [lanes * _TSTRIDE])
        for l in range(1, _L):
            s = s + plsc.load_gather(acc, [lanes * _TSTRIDE + l])
        out_v[pl.ds(g * _L, _L)] = 1.0 / (1.0 + jnp.exp(-s))
        return carry

    lax.fori_loop(0, _GROUPS, group, 0)
    pltpu.sync_copy(out_v, out_hbm.at[pl.ds(base, _BPW)])


@jax.jit
def _run(tw, cw, ttab, ctab):
    mesh = plsc.VectorSubcoreMesh(core_axis_name="c", subcore_axis_name="s")
    kern = functools.partial(
        pl.kernel,
        mesh=mesh,
        compiler_params=pltpu.CompilerParams(needs_layout_passes=False),
        out_type=jax.ShapeDtypeStruct((BATCH,), jnp.float32),
        scratch_types=[
            pltpu.VMEM((_BPW,), jnp.int32),
            pltpu.VMEM((_BPW,), jnp.int32),
            pltpu.VMEM((_L, 8, DIM), jnp.float32),
            pltpu.VMEM((_L, 8, DIM), jnp.float32),
            pltpu.VMEM((_L * _TSTRIDE,), jnp.float32),
            pltpu.VMEM((_BPW,), jnp.float32),
            pltpu.SemaphoreType.DMA,
            pltpu.SemaphoreType.DMA,
        ],
    )(_sc_body)
    return kern(tw, cw, ttab, ctab)


def kernel(target_word, context_word, target_table, context_table):
    tw = target_word.astype(jnp.int32)
    cw = context_word.astype(jnp.int32)
    return _run(tw, cw, target_table, context_table)
